# baseline (speedup 1.0000x reference)
import functools

import jax
import jax.numpy as jnp
from jax import lax
from jax.experimental import pallas as pl
from jax.experimental.pallas import tpu as pltpu
from jax.experimental.pallas import tpu_sc as plsc

N_TOKENS = 1000002
EMB_DIM = 64
PAD_DIM = 128
BATCH = 4096
HIST_LEN = 200

_INFO = plsc.get_sparse_core_info()
_NC = _INFO.num_cores
_NS = _INFO.num_subcores
_NW = _NC * _NS

_N = BATCH * HIST_LEN
_LANE = 128
_SUB = 8
_TD = EMB_DIM // _SUB
_TB = BATCH // _LANE
_ITEMS = HIST_LEN * _TB
_IPW = _ITEMS // _NW
_B_PER_W = _IPW * _LANE
_NBUF = 4
_N_GROUPS = _IPW // _NBUF


@jax.jit
def _gather_sc(x_flat2, table_p):
    mesh = plsc.VectorSubcoreMesh(core_axis_name="c", subcore_axis_name="s")

    @functools.partial(
        pl.kernel,
        mesh=mesh,
        out_type=jax.ShapeDtypeStruct(
            (HIST_LEN, _TD, _TB, _SUB, _LANE), jnp.float32
        ),
        scratch_types=[
            pltpu.VMEM((_B_PER_W,), jnp.int32),
            pltpu.VMEM((_NBUF, _LANE, EMB_DIM), jnp.float32),
            pltpu.VMEM((_NBUF, _TD, _SUB, _LANE), jnp.float32),
            pltpu.SemaphoreType.DMA((_NBUF,)),
            pltpu.SemaphoreType.DMA((_NBUF,)),
        ],
        compiler_params=pltpu.CompilerParams(
            use_tc_tiling_on_sc=False, needs_layout_passes=False
        ),
    )
    def k(x_hbm, table_hbm, out_hbm, idx_v, rows_v, t_v, gsem, ssem):
        wid = lax.axis_index("s") * _NC + lax.axis_index("c")
        base_id = wid * _IPW
        pltpu.sync_copy(x_hbm.at[pl.ds(wid * _B_PER_W, _B_PER_W)], idx_v)

        def group(g, carry):
            for b in range(_NBUF):
                i = g * _NBUF + b
                item = base_id + i
                h0 = item // _TB
                tb0 = item - h0 * _TB

                @pl.when(g > 0)
                def _wait_store():
                    pltpu.make_async_copy(
                        t_v.at[b],
                        out_hbm.at[h0, slice(None), tb0],
                        ssem.at[b],
                    ).wait()

                pltpu.make_async_copy(
                    table_hbm.at[idx_v.at[pl.ds(i * _LANE, _LANE)]],
                    rows_v.at[b],
                    gsem.at[b],
                ).start()
            for b in range(_NBUF):
                i = g * _NBUF + b
                item = base_id + i
                h = item // _TB
                tb = item - h * _TB
                pltpu.make_async_copy(
                    table_hbm.at[idx_v.at[pl.ds(i * _LANE, _LANE)]],
                    rows_v.at[b],
                    gsem.at[b],
                ).wait()
                def lg_body(lg, c2):
                    lane16 = lax.broadcasted_iota(jnp.int32, (16,), 0)
                    rows16 = rows_v.at[b, pl.ds(lg * 16, 16), slice(None)]
                    for td in range(_TD):
                        for s in range(_SUB):
                            col = jnp.full((16,), td * _SUB + s, jnp.int32)
                            vec = plsc.load_gather(rows16, [lane16, col])
                            t_v[b, td, s, pl.ds(lg * 16, 16)] = vec
                    return c2

                lax.fori_loop(0, _LANE // 16, lg_body, 0)
                pltpu.make_async_copy(
                    t_v.at[b],
                    out_hbm.at[h, slice(None), tb],
                    ssem.at[b],
                ).start()
            return carry

        lax.fori_loop(0, _N_GROUPS, group, 0)
        for b in range(_NBUF):
            i = (_N_GROUPS - 1) * _NBUF + b
            item = base_id + i
            h = item // _TB
            tb = item - h * _TB
            pltpu.make_async_copy(
                t_v.at[b],
                out_hbm.at[h, slice(None), tb],
                ssem.at[b],
            ).wait()

    return k(x_flat2, table_p)


def kernel(x, embedding_weight):
    x_flat2 = (x.astype(jnp.int32) * 2).T.reshape(-1)
    table_p = jnp.pad(
        embedding_weight, ((0, 0), (0, PAD_DIM - EMB_DIM))
    ).reshape(2 * N_TOKENS, EMB_DIM)
    out5 = _gather_sc(x_flat2, table_p)
    return out5.transpose(2, 4, 0, 1, 3).reshape(BATCH, HIST_LEN, EMB_DIM)


# speedup vs baseline: 2.1297x; 2.1297x over previous
import functools

import jax
import jax.numpy as jnp
from jax import lax
from jax.experimental import pallas as pl
from jax.experimental.pallas import tpu as pltpu
from jax.experimental.pallas import tpu_sc as plsc

N_TOKENS = 1000002
EMB_DIM = 64
PAD_DIM = 128
BATCH = 4096
HIST_LEN = 200

_INFO = plsc.get_sparse_core_info()
_NC = _INFO.num_cores
_NS = _INFO.num_subcores
_NW = _NC * _NS

_N = BATCH * HIST_LEN
_B_PER_W = _N // _NW
_CHUNK = 256
_NBUF = 4
_N_CHUNKS = _B_PER_W // _CHUNK
_N_GROUPS = _N_CHUNKS // _NBUF


@jax.jit
def _gather_sc(x_2d, table_p):
    mesh = plsc.VectorSubcoreMesh(core_axis_name="c", subcore_axis_name="s")

    @functools.partial(
        pl.kernel,
        mesh=mesh,
        out_type=jax.ShapeDtypeStruct((_N, PAD_DIM), jnp.float32),
        scratch_types=[
            pltpu.VMEM((_N_CHUNKS, _CHUNK), jnp.int32),
            pltpu.VMEM((_NBUF, _CHUNK, EMB_DIM), jnp.float32),
            pltpu.SemaphoreType.DMA((_NBUF,)),
            pltpu.SemaphoreType.DMA((_NBUF,)),
        ],
        compiler_params=pltpu.CompilerParams(use_tc_tiling_on_sc=False),
    )
    def k(x_hbm, table_hbm, out_hbm, idx_v, rows_v, gsem, ssem):
        wid = lax.axis_index("s") * _NC + lax.axis_index("c")
        base = wid * _B_PER_W
        pltpu.sync_copy(
            x_hbm.at[pl.ds(wid * _N_CHUNKS, _N_CHUNKS), :], idx_v
        )

        def group(g, carry):
            for b in range(_NBUF):
                i = g * _NBUF + b

                @pl.when(g > 0)
                def _wait_store():
                    pltpu.make_async_copy(
                        rows_v.at[b],
                        out_hbm.at[pl.ds(base, _CHUNK), pl.ds(0, EMB_DIM)],
                        ssem.at[b],
                    ).wait()

                pltpu.make_async_copy(
                    table_hbm.at[idx_v.at[i]],
                    rows_v.at[b],
                    gsem.at[b],
                ).start()
            for b in range(_NBUF):
                i = g * _NBUF + b
                start = base + i * _CHUNK
                pltpu.make_async_copy(
                    table_hbm.at[idx_v.at[i]],
                    rows_v.at[b],
                    gsem.at[b],
                ).wait()
                pltpu.make_async_copy(
                    rows_v.at[b],
                    out_hbm.at[pl.ds(start, _CHUNK), pl.ds(0, EMB_DIM)],
                    ssem.at[b],
                ).start()
            return carry

        lax.fori_loop(0, _N_GROUPS, group, 0)
        for b in range(_NBUF):
            pltpu.make_async_copy(
                rows_v.at[b],
                out_hbm.at[pl.ds(base, _CHUNK), pl.ds(0, EMB_DIM)],
                ssem.at[b],
            ).wait()

    return k(x_2d, table_p)


def kernel(x, embedding_weight):
    x_2d = (x.astype(jnp.int32) * 2).reshape(_N // _CHUNK, _CHUNK)
    table_p = jnp.pad(
        embedding_weight, ((0, 0), (0, PAD_DIM - EMB_DIM))
    ).reshape(2 * N_TOKENS, EMB_DIM)
    out_p = _gather_sc(x_2d, table_p)
    return out_p[:, :EMB_DIM].reshape(BATCH, HIST_LEN, EMB_DIM)


# speedup vs baseline: 2.1350x; 1.0025x over previous
import functools

import jax
import jax.numpy as jnp
from jax import lax
from jax.experimental import pallas as pl
from jax.experimental.pallas import tpu as pltpu
from jax.experimental.pallas import tpu_sc as plsc

N_TOKENS = 1000002
EMB_DIM = 64
PAD_DIM = 128
BATCH = 4096
HIST_LEN = 200

_INFO = plsc.get_sparse_core_info()
_NC = _INFO.num_cores
_NS = _INFO.num_subcores
_NW = _NC * _NS

_N = BATCH * HIST_LEN
_B_PER_W = _N // _NW
_CHUNK = 128
_NBUF = 4
_N_CHUNKS = _B_PER_W // _CHUNK
_N_GROUPS = _N_CHUNKS // _NBUF


@jax.jit
def _gather_sc(x_2d, table_p):
    mesh = plsc.VectorSubcoreMesh(core_axis_name="c", subcore_axis_name="s")

    @functools.partial(
        pl.kernel,
        mesh=mesh,
        out_type=jax.ShapeDtypeStruct((_N, PAD_DIM), jnp.float32),
        scratch_types=[
            pltpu.VMEM((_N_CHUNKS, _CHUNK), jnp.int32),
            pltpu.VMEM((_NBUF, _CHUNK, EMB_DIM), jnp.float32),
            pltpu.SemaphoreType.DMA((_NBUF,)),
            pltpu.SemaphoreType.DMA((_NBUF,)),
        ],
        compiler_params=pltpu.CompilerParams(use_tc_tiling_on_sc=False),
    )
    def k(x_hbm, table_hbm, out_hbm, idx_v, rows_v, gsem, ssem):
        wid = lax.axis_index("s") * _NC + lax.axis_index("c")
        base = wid * _B_PER_W
        pltpu.sync_copy(
            x_hbm.at[pl.ds(wid * _N_CHUNKS, _N_CHUNKS), :], idx_v
        )

        def group(g, carry):
            for b in range(_NBUF):
                i = g * _NBUF + b

                @pl.when(g > 0)
                def _wait_store():
                    pltpu.make_async_copy(
                        rows_v.at[b],
                        out_hbm.at[pl.ds(base, _CHUNK), pl.ds(0, EMB_DIM)],
                        ssem.at[b],
                    ).wait()

                pltpu.make_async_copy(
                    table_hbm.at[idx_v.at[i]],
                    rows_v.at[b],
                    gsem.at[b],
                ).start()
            for b in range(_NBUF):
                i = g * _NBUF + b
                start = base + i * _CHUNK
                pltpu.make_async_copy(
                    table_hbm.at[idx_v.at[i]],
                    rows_v.at[b],
                    gsem.at[b],
                ).wait()
                pltpu.make_async_copy(
                    rows_v.at[b],
                    out_hbm.at[pl.ds(start, _CHUNK), pl.ds(0, EMB_DIM)],
                    ssem.at[b],
                ).start()
            return carry

        lax.fori_loop(0, _N_GROUPS, group, 0)
        for b in range(_NBUF):
            pltpu.make_async_copy(
                rows_v.at[b],
                out_hbm.at[pl.ds(base, _CHUNK), pl.ds(0, EMB_DIM)],
                ssem.at[b],
            ).wait()

    return k(x_2d, table_p)


def kernel(x, embedding_weight):
    x_2d = (x.astype(jnp.int32) * 2).reshape(_N // _CHUNK, _CHUNK)
    table_p = jnp.pad(
        embedding_weight, ((0, 0), (0, PAD_DIM - EMB_DIM))
    ).reshape(2 * N_TOKENS, EMB_DIM)
    out_p = _gather_sc(x_2d, table_p)
    return out_p[:, :EMB_DIM].reshape(BATCH, HIST_LEN, EMB_DIM)
